# baseline (speedup 1.0000x reference)
import functools

import jax
import jax.numpy as jnp
from jax import lax
from jax.experimental import pallas as pl
from jax.experimental.pallas import tpu as pltpu
from jax.experimental.pallas import tpu_sc as plsc

_VOCAB = 100000
_B = 1024
_T = 200
_D = 512
_N = _B * _T
_BLOCKS = (256, 320, 384, 64)
_OFFS = tuple(sum(_BLOCKS[:k]) for k in range(len(_BLOCKS)))
_NW = 32
_C = 16


def _pe_table():
    x = jnp.arange(_T, dtype=jnp.float32)[:, None]
    y = jnp.arange(_D, dtype=jnp.float32)[None, :]
    temp = jnp.power(10000.0, 2.0 * y / _D).astype(jnp.float32)
    s = jnp.sin(x / temp)
    c = jnp.cos(x / temp)
    z = jnp.zeros((_T, _D), dtype=jnp.float32)
    z = z.at[:, 0::2].set(s[:, 0::2])
    z = z.at[:, 1::2].set(c[:, 1::2])
    return z


def _sc_gather_body(n_seqs, n_out_chunks):
    per_w = n_seqs * _T // _NW
    nch = per_w // _C
    nbuf = 10 if nch % 10 == 0 else 5

    def body(table_hbm, idx_hbm, out_hbm, idx_v, *rest):
        bufs = rest[:nbuf]
        in_sems = rest[nbuf:2 * nbuf]
        out_sems = rest[2 * nbuf:3 * nbuf]

        cid = lax.axis_index("c")
        sid = lax.axis_index("s")
        wid = sid * 2 + cid

        pltpu.sync_copy(idx_hbm.at[pl.ds(wid * per_w, per_w)], idx_v)

        def start_in(c, b):
            pltpu.make_async_copy(
                table_hbm.at[idx_v.at[pl.ds(c * _C, _C)]], bufs[b], in_sems[b]
            ).start()

        def wait_in(b):
            pltpu.make_async_copy(
                table_hbm.at[idx_v.at[pl.ds(0, _C)]], bufs[b], in_sems[b]
            ).wait()

        def start_out(c, b):
            pltpu.make_async_copy(
                bufs[b], out_hbm.at[wid * nch + c], out_sems[b]
            ).start()

        def wait_out(b):
            pltpu.make_async_copy(
                bufs[b], out_hbm.at[0], out_sems[b]
            ).wait()

        for b in range(nbuf):
            start_in(b, b)

        @pl.loop(0, nch, step=nbuf)
        def _chunks(c0):
            for b in range(nbuf):
                c = c0 + b
                wait_in(b)
                start_out(c, b)

                @pl.when(c + nbuf < nch)
                def _prefetch():
                    wait_out(b)
                    start_in(c + nbuf, b)

        for b in range(nbuf):
            wait_out(b)

    return pl.kernel(
        body,
        out_type=jax.ShapeDtypeStruct((n_out_chunks, _C, _D), jnp.float32),
        mesh=plsc.VectorSubcoreMesh(core_axis_name="c", subcore_axis_name="s"),
        scratch_types=[
            pltpu.VMEM((per_w,), jnp.int32),
        ] + [pltpu.VMEM((_C, _D), jnp.float32)] * nbuf
          + [pltpu.SemaphoreType.DMA] * (2 * nbuf),
    )


_SPB = 8
_TR = _SPB * _T


def _tc_body(g_ref, pe_ref, o_ref):
    for j in range(_SPB):
        sl = pl.ds(j * _T, _T)
        o_ref[sl, :] = g_ref[sl, :] + pe_ref[...]


def _tc_add_first(acc, pe, n_seqs):
    def body(x_ref, pe_ref, o_ref):
        _tc_body(x_ref, pe_ref, o_ref)

    return pl.pallas_call(
        body,
        grid=(n_seqs // _SPB,),
        in_specs=[
            pl.BlockSpec((_TR, _D), lambda i: (i, 0)),
            pl.BlockSpec((_T, _D), lambda i: (0, 0)),
        ],
        out_specs=pl.BlockSpec((_TR, _D), lambda i: (i, 0)),
        out_shape=jax.ShapeDtypeStruct((_N, _D), jnp.float32),
        input_output_aliases={0: 0},
    )(acc, pe)


def _tc_add_block(off_seqs, n_seqs):
    def body(acc_ref, g_ref, pe_ref, o_ref):
        _tc_body(g_ref, pe_ref, o_ref)

    blk0 = off_seqs // _SPB
    return pl.pallas_call(
        body,
        grid=(n_seqs // _SPB,),
        in_specs=[
            pl.BlockSpec((8, 128), lambda i: (0, 0)),
            pl.BlockSpec((_TR, _D), lambda i: (i, 0)),
            pl.BlockSpec((_T, _D), lambda i: (0, 0)),
        ],
        out_specs=pl.BlockSpec((_TR, _D), lambda i: (blk0 + i, 0)),
        out_shape=jax.ShapeDtypeStruct((_N, _D), jnp.float32),
        input_output_aliases={0: 0},
    )


@functools.partial(jax.jit, static_argnums=())
def _run(table, idx, pe):
    s0 = _BLOCKS[0]
    gather0 = _sc_gather_body(s0, _N // _C)
    acc = gather0(table, idx[: s0 * _T]).reshape(_N, _D)
    parts = []
    for k in range(1, len(_BLOCKS)):
        o, s = _OFFS[k], _BLOCKS[k]
        g = _sc_gather_body(s, s * _T // _C)(
            table, idx[o * _T:(o + s) * _T])
        parts.append(g.reshape(s * _T, _D))

    acc = _tc_add_first(acc, pe, s0)
    for k in range(1, len(_BLOCKS)):
        acc = _tc_add_block(_OFFS[k], _BLOCKS[k])(acc, parts[k - 1], pe)
    return acc


def kernel(X, table):
    idx = X.reshape(-1).astype(jnp.int32)
    pe = _pe_table()
    out = _run(table, idx, pe)
    return out.reshape(_B, _T, _D)


# speedup vs baseline: 1.0169x; 1.0169x over previous
import functools

import jax
import jax.numpy as jnp
from jax import lax
from jax.experimental import pallas as pl
from jax.experimental.pallas import tpu as pltpu
from jax.experimental.pallas import tpu_sc as plsc

_VOCAB = 100000
_B = 1024
_T = 200
_D = 512
_N = _B * _T
_BLOCKS = (320, 320, 320, 64)
_OFFS = tuple(sum(_BLOCKS[:k]) for k in range(len(_BLOCKS)))
_NW = 32
_C = 16


def _pe_table():
    x = jnp.arange(_T, dtype=jnp.float32)[:, None]
    y = jnp.arange(_D, dtype=jnp.float32)[None, :]
    temp = jnp.power(10000.0, 2.0 * y / _D).astype(jnp.float32)
    s = jnp.sin(x / temp)
    c = jnp.cos(x / temp)
    z = jnp.zeros((_T, _D), dtype=jnp.float32)
    z = z.at[:, 0::2].set(s[:, 0::2])
    z = z.at[:, 1::2].set(c[:, 1::2])
    return z


def _sc_gather_body(n_seqs, n_out_chunks):
    per_w = n_seqs * _T // _NW
    nch = per_w // _C
    nbuf = 10 if nch % 10 == 0 else 5

    def body(table_hbm, idx_hbm, out_hbm, idx_v, *rest):
        bufs = rest[:nbuf]
        in_sems = rest[nbuf:2 * nbuf]
        out_sems = rest[2 * nbuf:3 * nbuf]

        cid = lax.axis_index("c")
        sid = lax.axis_index("s")
        wid = sid * 2 + cid

        pltpu.sync_copy(idx_hbm.at[pl.ds(wid * per_w, per_w)], idx_v)

        def start_in(c, b):
            pltpu.make_async_copy(
                table_hbm.at[idx_v.at[pl.ds(c * _C, _C)]], bufs[b], in_sems[b]
            ).start()

        def wait_in(b):
            pltpu.make_async_copy(
                table_hbm.at[idx_v.at[pl.ds(0, _C)]], bufs[b], in_sems[b]
            ).wait()

        def start_out(c, b):
            pltpu.make_async_copy(
                bufs[b], out_hbm.at[wid * nch + c], out_sems[b]
            ).start()

        def wait_out(b):
            pltpu.make_async_copy(
                bufs[b], out_hbm.at[0], out_sems[b]
            ).wait()

        for b in range(nbuf):
            start_in(b, b)

        @pl.loop(0, nch, step=nbuf)
        def _chunks(c0):
            for b in range(nbuf):
                c = c0 + b
                wait_in(b)
                start_out(c, b)

                @pl.when(c + nbuf < nch)
                def _prefetch():
                    wait_out(b)
                    start_in(c + nbuf, b)

        for b in range(nbuf):
            wait_out(b)

    return pl.kernel(
        body,
        out_type=jax.ShapeDtypeStruct((n_out_chunks, _C, _D), jnp.float32),
        mesh=plsc.VectorSubcoreMesh(core_axis_name="c", subcore_axis_name="s"),
        scratch_types=[
            pltpu.VMEM((per_w,), jnp.int32),
        ] + [pltpu.VMEM((_C, _D), jnp.float32)] * nbuf
          + [pltpu.SemaphoreType.DMA] * (2 * nbuf),
    )


_SPB = 16
_TR = _SPB * _T


def _tc_body(g_ref, pe_ref, o_ref):
    for j in range(_SPB):
        sl = pl.ds(j * _T, _T)
        o_ref[sl, :] = g_ref[sl, :] + pe_ref[...]


def _tc_add_first(acc, pe, n_seqs):
    def body(x_ref, pe_ref, o_ref):
        _tc_body(x_ref, pe_ref, o_ref)

    return pl.pallas_call(
        body,
        grid=(n_seqs // _SPB,),
        in_specs=[
            pl.BlockSpec((_TR, _D), lambda i: (i, 0)),
            pl.BlockSpec((_T, _D), lambda i: (0, 0)),
        ],
        out_specs=pl.BlockSpec((_TR, _D), lambda i: (i, 0)),
        out_shape=jax.ShapeDtypeStruct((_N, _D), jnp.float32),
        input_output_aliases={0: 0},
    )(acc, pe)


def _tc_add_block(off_seqs, n_seqs):
    def body(acc_ref, g_ref, pe_ref, o_ref):
        _tc_body(g_ref, pe_ref, o_ref)

    blk0 = off_seqs // _SPB
    return pl.pallas_call(
        body,
        grid=(n_seqs // _SPB,),
        in_specs=[
            pl.BlockSpec((8, 128), lambda i: (0, 0)),
            pl.BlockSpec((_TR, _D), lambda i: (i, 0)),
            pl.BlockSpec((_T, _D), lambda i: (0, 0)),
        ],
        out_specs=pl.BlockSpec((_TR, _D), lambda i: (blk0 + i, 0)),
        out_shape=jax.ShapeDtypeStruct((_N, _D), jnp.float32),
        input_output_aliases={0: 0},
    )


@functools.partial(jax.jit, static_argnums=())
def _run(table, idx, pe):
    s0 = _BLOCKS[0]
    gather0 = _sc_gather_body(s0, _N // _C)
    acc = gather0(table, idx[: s0 * _T]).reshape(_N, _D)
    parts = []
    for k in range(1, len(_BLOCKS)):
        o, s = _OFFS[k], _BLOCKS[k]
        g = _sc_gather_body(s, s * _T // _C)(
            table, idx[o * _T:(o + s) * _T])
        parts.append(g.reshape(s * _T, _D))

    acc = _tc_add_first(acc, pe, s0)
    for k in range(1, len(_BLOCKS)):
        acc = _tc_add_block(_OFFS[k], _BLOCKS[k])(acc, parts[k - 1], pe)
    return acc


def kernel(X, table):
    idx = X.reshape(-1).astype(jnp.int32)
    pe = _pe_table()
    out = _run(table, idx, pe)
    return out.reshape(_B, _T, _D)


# speedup vs baseline: 1.0207x; 1.0037x over previous
import functools

import jax
import jax.numpy as jnp
from jax import lax
from jax.experimental import pallas as pl
from jax.experimental.pallas import tpu as pltpu
from jax.experimental.pallas import tpu_sc as plsc

_VOCAB = 100000
_B = 1024
_T = 200
_D = 512
_N = _B * _T
_BLOCKS = (320, 320, 320, 64)
_OFFS = tuple(sum(_BLOCKS[:k]) for k in range(len(_BLOCKS)))
_NW = 32
_C = 16


def _pe_table():
    x = jnp.arange(_T, dtype=jnp.float32)[:, None]
    y = jnp.arange(_D, dtype=jnp.float32)[None, :]
    temp = jnp.power(10000.0, 2.0 * y / _D).astype(jnp.float32)
    s = jnp.sin(x / temp)
    c = jnp.cos(x / temp)
    z = jnp.zeros((_T, _D), dtype=jnp.float32)
    z = z.at[:, 0::2].set(s[:, 0::2])
    z = z.at[:, 1::2].set(c[:, 1::2])
    return z


def _sc_gather_body(n_seqs, n_out_chunks):
    per_w = n_seqs * _T // _NW
    nch = per_w // _C
    nbuf = 10 if nch % 10 == 0 else 5

    def body(table_hbm, idx_hbm, out_hbm, idx_v, *rest):
        bufs = rest[:nbuf]
        in_sems = rest[nbuf:2 * nbuf]
        out_sems = rest[2 * nbuf:3 * nbuf]

        cid = lax.axis_index("c")
        sid = lax.axis_index("s")
        wid = sid * 2 + cid

        pltpu.sync_copy(idx_hbm.at[pl.ds(wid * per_w, per_w)], idx_v)

        def start_in(c, b):
            pltpu.make_async_copy(
                table_hbm.at[idx_v.at[pl.ds(c * _C, _C)]], bufs[b], in_sems[b]
            ).start()

        def wait_in(b):
            pltpu.make_async_copy(
                table_hbm.at[idx_v.at[pl.ds(0, _C)]], bufs[b], in_sems[b]
            ).wait()

        def start_out(c, b):
            pltpu.make_async_copy(
                bufs[b], out_hbm.at[wid * nch + c], out_sems[b]
            ).start()

        def wait_out(b):
            pltpu.make_async_copy(
                bufs[b], out_hbm.at[0], out_sems[b]
            ).wait()

        for b in range(nbuf):
            start_in(b, b)

        @pl.loop(0, nch, step=nbuf)
        def _chunks(c0):
            for b in range(nbuf):
                c = c0 + b
                wait_in(b)
                start_out(c, b)

                @pl.when(c + nbuf < nch)
                def _prefetch():
                    wait_out(b)
                    start_in(c + nbuf, b)

        for b in range(nbuf):
            wait_out(b)

    return pl.kernel(
        body,
        out_type=jax.ShapeDtypeStruct((n_out_chunks, _C, _D), jnp.float32),
        mesh=plsc.VectorSubcoreMesh(core_axis_name="c", subcore_axis_name="s"),
        scratch_types=[
            pltpu.VMEM((per_w,), jnp.int32),
        ] + [pltpu.VMEM((_C, _D), jnp.float32)] * nbuf
          + [pltpu.SemaphoreType.DMA] * (2 * nbuf),
    )


_SPB = 32
_TR = _SPB * _T


def _tc_body(g_ref, pe_ref, o_ref):
    for j in range(_SPB):
        sl = pl.ds(j * _T, _T)
        o_ref[sl, :] = g_ref[sl, :] + pe_ref[...]


def _tc_add_first(acc, pe, n_seqs):
    def body(x_ref, pe_ref, o_ref):
        _tc_body(x_ref, pe_ref, o_ref)

    return pl.pallas_call(
        body,
        grid=(n_seqs // _SPB,),
        in_specs=[
            pl.BlockSpec((_TR, _D), lambda i: (i, 0)),
            pl.BlockSpec((_T, _D), lambda i: (0, 0)),
        ],
        out_specs=pl.BlockSpec((_TR, _D), lambda i: (i, 0)),
        out_shape=jax.ShapeDtypeStruct((_N, _D), jnp.float32),
        input_output_aliases={0: 0},
    )(acc, pe)


def _tc_add_block(off_seqs, n_seqs):
    def body(acc_ref, g_ref, pe_ref, o_ref):
        _tc_body(g_ref, pe_ref, o_ref)

    blk0 = off_seqs // _SPB
    return pl.pallas_call(
        body,
        grid=(n_seqs // _SPB,),
        in_specs=[
            pl.BlockSpec((8, 128), lambda i: (0, 0)),
            pl.BlockSpec((_TR, _D), lambda i: (i, 0)),
            pl.BlockSpec((_T, _D), lambda i: (0, 0)),
        ],
        out_specs=pl.BlockSpec((_TR, _D), lambda i: (blk0 + i, 0)),
        out_shape=jax.ShapeDtypeStruct((_N, _D), jnp.float32),
        input_output_aliases={0: 0},
    )


@functools.partial(jax.jit, static_argnums=())
def _run(table, idx, pe):
    s0 = _BLOCKS[0]
    gather0 = _sc_gather_body(s0, _N // _C)
    acc = gather0(table, idx[: s0 * _T]).reshape(_N, _D)
    parts = []
    for k in range(1, len(_BLOCKS)):
        o, s = _OFFS[k], _BLOCKS[k]
        g = _sc_gather_body(s, s * _T // _C)(
            table, idx[o * _T:(o + s) * _T])
        parts.append(g.reshape(s * _T, _D))

    acc = _tc_add_first(acc, pe, s0)
    for k in range(1, len(_BLOCKS)):
        acc = _tc_add_block(_OFFS[k], _BLOCKS[k])(acc, parts[k - 1], pe)
    return acc


def kernel(X, table):
    idx = X.reshape(-1).astype(jnp.int32)
    pe = _pe_table()
    out = _run(table, idx, pe)
    return out.reshape(_B, _T, _D)
